# baseline (speedup 1.0000x reference)
import functools

import jax
import jax.numpy as jnp
from jax import lax
from jax.experimental import pallas as pl
from jax.experimental.pallas import tpu as pltpu
from jax.experimental.pallas import tpu_sc as plsc

N = 10000
E = 320000
D_IN = 128
H1 = 32
H2 = 16

NC = 2
NS = 16
NW = NC * NS
EPW = E // NW
CH = 400
NCH = EPW // CH
NPAD = 10240
RPT = NPAD // NS


def _spmm_sc(h, src_f, dst_f, wb_f):
  mesh = plsc.VectorSubcoreMesh(core_axis_name="c", subcore_axis_name="s")

  @functools.partial(
      pl.kernel,
      out_type=jax.ShapeDtypeStruct((NC, NPAD, H1), jnp.float32),
      mesh=mesh,
      compiler_params=pltpu.CompilerParams(
          use_tc_tiling_on_sc=False, needs_layout_passes=False),
      scratch_types=[
          [pltpu.VMEM((CH,), jnp.int32)] * 2,
          [pltpu.VMEM((CH,), jnp.int32)] * 2,
          [pltpu.VMEM((CH, H1), jnp.float32)] * 2,
          [pltpu.VMEM((CH * 16,), jnp.float32)] * 2,
          pltpu.VMEM((CH, H1), jnp.float32),
          pltpu.VMEM((RPT, H1), jnp.float32),
          pltpu.VMEM_SHARED((NPAD, H1), jnp.float32),
          [pltpu.SemaphoreType.DMA] * 2,
          [pltpu.SemaphoreType.DMA] * 2,
      ],
  )
  def k(h_hbm, src_hbm, dst_hbm, wb_hbm, out_hbm,
        src_b, dst_b, rows_b, wb_b, wrows, stage_v, acc_sh, sems_e, sems_r):
    cid = lax.axis_index("c")
    sid = lax.axis_index("s")
    wid = cid * NS + sid
    base = wid * EPW

    zero16 = jnp.zeros((16,), jnp.float32)

    def zbody(i, carry):
      stage_v[i, pl.ds(0, 16)] = zero16
      stage_v[i, pl.ds(16, 16)] = zero16
      return carry

    lax.fori_loop(0, RPT, zbody, 0)
    pltpu.sync_copy(stage_v, acc_sh.at[pl.ds(sid * RPT, RPT)])

    def start_edges(c, par):
      off = base + c * CH
      pltpu.async_copy(src_hbm.at[pl.ds(off, CH)], src_b[par], sems_e[par])
      pltpu.async_copy(dst_hbm.at[pl.ds(off, CH)], dst_b[par], sems_e[par])
      pltpu.async_copy(
          wb_hbm.at[pl.ds(off * 16, CH * 16)], wb_b[par], sems_e[par])

    def wait_edges(c, par):
      off = base + c * CH
      pltpu.make_async_copy(
          src_hbm.at[pl.ds(off, CH)], src_b[par], sems_e[par]).wait()
      pltpu.make_async_copy(
          dst_hbm.at[pl.ds(off, CH)], dst_b[par], sems_e[par]).wait()
      pltpu.make_async_copy(
          wb_hbm.at[pl.ds(off * 16, CH * 16)], wb_b[par], sems_e[par]).wait()

    def start_rows(par):
      pltpu.async_copy(h_hbm.at[src_b[par]], rows_b[par], sems_r[par])

    def wait_rows(par):
      pltpu.make_async_copy(
          h_hbm.at[src_b[par]], rows_b[par], sems_r[par]).wait()

    def scale(par):
      def sbody(e, carry):
        wv = wb_b[par][pl.ds(e * 16, 16)]
        wrows[e, pl.ds(0, 16)] = rows_b[par][e, pl.ds(0, 16)] * wv
        wrows[e, pl.ds(16, 16)] = rows_b[par][e, pl.ds(16, 16)] * wv
        return carry

      lax.fori_loop(0, CH, sbody, 0, unroll=8)

    start_edges(0, 0)
    start_edges(1, 1)
    wait_edges(0, 0)
    start_rows(0)

    def kbody(kk, carry):
      for par in range(2):
        c = 2 * kk + par

        @pl.when(c < NCH)
        def _():
          wait_rows(par)

          @pl.when(c + 1 < NCH)
          def _():
            wait_edges(c + 1, 1 - par)
            start_rows(1 - par)

          scale(par)
          pltpu.sync_copy(wrows, acc_sh.at[dst_b[par]], add=True)

          @pl.when(c + 2 < NCH)
          def _():
            start_edges(c + 2, par)

      return carry

    lax.fori_loop(0, (NCH + 1) // 2, kbody, 0)
    plsc.subcore_barrier()

    pltpu.sync_copy(acc_sh.at[pl.ds(sid * RPT, RPT)], stage_v)
    pltpu.sync_copy(stage_v, out_hbm.at[cid, pl.ds(sid * RPT, RPT)])

  return k(h, src_f, dst_f, wb_f)


def _mm1_tc(x, W1):
  def body(x_ref, w_ref, o_ref):
    o_ref[...] = jnp.dot(x_ref[...], w_ref[...],
                         preferred_element_type=jnp.float32)

  return pl.pallas_call(
      body,
      grid=(10,),
      in_specs=[
          pl.BlockSpec((N // 10, D_IN), lambda i: (i, 0)),
          pl.BlockSpec((D_IN, H1), lambda i: (0, 0)),
      ],
      out_specs=pl.BlockSpec((N // 10, H1), lambda i: (i, 0)),
      out_shape=jax.ShapeDtypeStruct((N, H1), jnp.float32),
  )(x, W1)


def _relu_combine_tc(p):
  def body(p_ref, o_ref):
    o_ref[...] = jnp.maximum(p_ref[0] + p_ref[1], 0.0)

  blk = NPAD // 10
  return pl.pallas_call(
      body,
      grid=(10,),
      in_specs=[pl.BlockSpec((NC, blk, H1), lambda i: (0, i, 0))],
      out_specs=pl.BlockSpec((blk, H1), lambda i: (i, 0)),
      out_shape=jax.ShapeDtypeStruct((NPAD, H1), jnp.float32),
  )(p)


def _finalize_tc(q, eps, W2, W3):
  def body(q_ref, eps_ref, w2_ref, w3_ref, mu_ref, lv_ref, z_ref):
    s = q_ref[0] + q_ref[1]
    mu = jnp.dot(s, w2_ref[...], preferred_element_type=jnp.float32)
    lv = jnp.dot(s, w3_ref[...], preferred_element_type=jnp.float32)
    mu_ref[...] = mu
    lv_ref[...] = lv
    z_ref[...] = eps_ref[...] * jnp.exp(lv) + mu

  blk = NPAD // 10
  return pl.pallas_call(
      body,
      grid=(10,),
      in_specs=[
          pl.BlockSpec((NC, blk, H1), lambda i: (0, i, 0)),
          pl.BlockSpec((blk, H2), lambda i: (i, 0)),
          pl.BlockSpec((H1, H2), lambda i: (0, 0)),
          pl.BlockSpec((H1, H2), lambda i: (0, 0)),
      ],
      out_specs=[
          pl.BlockSpec((blk, H2), lambda i: (i, 0)),
          pl.BlockSpec((blk, H2), lambda i: (i, 0)),
          pl.BlockSpec((blk, H2), lambda i: (i, 0)),
      ],
      out_shape=[
          jax.ShapeDtypeStruct((N, H2), jnp.float32),
          jax.ShapeDtypeStruct((N, H2), jnp.float32),
          jax.ShapeDtypeStruct((N, H2), jnp.float32),
      ],
  )(q, eps, W2, W3)


def _gram_tc(z):
  BR = 256
  gr = pl.cdiv(N, BR)

  def body(zr_ref, zc_ref, o_ref):
    o_ref[...] = lax.dot_general(
        zr_ref[...], zc_ref[...],
        (((1,), (1,)), ((), ())),
        preferred_element_type=jnp.float32)

  return pl.pallas_call(
      body,
      grid=(gr,),
      in_specs=[
          pl.BlockSpec((BR, H2), lambda i: (i, 0)),
          pl.BlockSpec((N, H2), lambda i: (0, 0)),
      ],
      out_specs=pl.BlockSpec((BR, N), lambda i: (i, 0)),
      out_shape=jax.ShapeDtypeStruct((N, N), jnp.float32),
  )(z, z)


def kernel(x, edge_index, edge_weight, eps, W1, W2, W3):
  src = edge_index[0].astype(jnp.int32)
  dst = edge_index[1].astype(jnp.int32)
  wb = jnp.broadcast_to(edge_weight[:, None], (E, 16)).reshape(E * 16)

  t1 = _mm1_tc(x, W1)
  p = _spmm_sc(t1, src, dst, wb)
  h1 = _relu_combine_tc(p)
  q = _spmm_sc(h1, src, dst, wb)
  mu, logvar, z = _finalize_tc(q, eps, W2, W3)
  adj = _gram_tc(z)
  return (adj, mu, logvar)


# speedup vs baseline: 1.3735x; 1.3735x over previous
import functools

import jax
import jax.numpy as jnp
from jax import lax
from jax.experimental import pallas as pl
from jax.experimental.pallas import tpu as pltpu
from jax.experimental.pallas import tpu_sc as plsc

N = 10000
E = 320000
D_IN = 128
H1 = 32
H2 = 16

NC = 2
NS = 16
NW = NC * NS
EPW = E // NW
CH = 400
NCH = EPW // CH
NPAD = 10240
RPT = NPAD // NS


WR = CH * 16 // 128


def _spmm_sc(h, ei, wb2):
  mesh = plsc.VectorSubcoreMesh(core_axis_name="c", subcore_axis_name="s")

  @functools.partial(
      pl.kernel,
      out_type=jax.ShapeDtypeStruct((NC, NPAD, H1), jnp.float32),
      mesh=mesh,
      compiler_params=pltpu.CompilerParams(
          use_tc_tiling_on_sc=False, needs_layout_passes=False),
      scratch_types=[
          [pltpu.VMEM((CH,), jnp.int32)] * 2,
          [pltpu.VMEM((CH,), jnp.int32)] * 2,
          [pltpu.VMEM((CH, H1), jnp.float32)] * 2,
          [pltpu.VMEM((WR, 128), jnp.float32)] * 2,
          pltpu.VMEM((CH, H1), jnp.float32),
          pltpu.VMEM((RPT, H1), jnp.float32),
          pltpu.VMEM_SHARED((NPAD, H1), jnp.float32),
          [pltpu.SemaphoreType.DMA] * 2,
          [pltpu.SemaphoreType.DMA] * 2,
      ],
  )
  def k(h_hbm, ei_hbm, wb_hbm, out_hbm,
        src_b, dst_b, rows_b, wb_b, wrows, stage_v, acc_sh, sems_e, sems_r):
    cid = lax.axis_index("c")
    sid = lax.axis_index("s")
    wid = cid * NS + sid
    base = wid * EPW

    zero16 = jnp.zeros((16,), jnp.float32)

    def zbody(i, carry):
      stage_v[i, pl.ds(0, 16)] = zero16
      stage_v[i, pl.ds(16, 16)] = zero16
      return carry

    lax.fori_loop(0, RPT, zbody, 0)
    pltpu.sync_copy(stage_v, acc_sh.at[pl.ds(sid * RPT, RPT)])

    def start_edges(c, par):
      off = base + c * CH
      wroff = off * 16 // 128
      pltpu.async_copy(ei_hbm.at[0, pl.ds(off, CH)], src_b[par], sems_e[par])
      pltpu.async_copy(ei_hbm.at[1, pl.ds(off, CH)], dst_b[par], sems_e[par])
      pltpu.async_copy(
          wb_hbm.at[pl.ds(wroff, WR)], wb_b[par], sems_e[par])

    def wait_edges(c, par):
      off = base + c * CH
      wroff = off * 16 // 128
      pltpu.make_async_copy(
          ei_hbm.at[0, pl.ds(off, CH)], src_b[par], sems_e[par]).wait()
      pltpu.make_async_copy(
          ei_hbm.at[1, pl.ds(off, CH)], dst_b[par], sems_e[par]).wait()
      pltpu.make_async_copy(
          wb_hbm.at[pl.ds(wroff, WR)], wb_b[par], sems_e[par]).wait()

    def start_rows(par):
      pltpu.async_copy(h_hbm.at[src_b[par]], rows_b[par], sems_r[par])

    def wait_rows(par):
      pltpu.make_async_copy(
          h_hbm.at[src_b[par]], rows_b[par], sems_r[par]).wait()

    def scale(par):
      def sbody(r, carry):
        for l8 in range(8):
          e = r * 8 + l8
          wv = wb_b[par][r, pl.ds(l8 * 16, 16)]
          wrows[e, pl.ds(0, 16)] = rows_b[par][e, pl.ds(0, 16)] * wv
          wrows[e, pl.ds(16, 16)] = rows_b[par][e, pl.ds(16, 16)] * wv
        return carry

      lax.fori_loop(0, WR, sbody, 0, unroll=2)

    start_edges(0, 0)
    start_edges(1, 1)
    wait_edges(0, 0)
    start_rows(0)

    def kbody(kk, carry):
      for par in range(2):
        c = 2 * kk + par

        @pl.when(c < NCH)
        def _():
          wait_rows(par)

          @pl.when(c + 1 < NCH)
          def _():
            wait_edges(c + 1, 1 - par)
            start_rows(1 - par)

          scale(par)
          pltpu.sync_copy(wrows, acc_sh.at[dst_b[par]], add=True)

          @pl.when(c + 2 < NCH)
          def _():
            start_edges(c + 2, par)

      return carry

    lax.fori_loop(0, (NCH + 1) // 2, kbody, 0)
    plsc.subcore_barrier()

    pltpu.sync_copy(acc_sh.at[pl.ds(sid * RPT, RPT)], stage_v)
    pltpu.sync_copy(stage_v, out_hbm.at[cid, pl.ds(sid * RPT, RPT)])

  return k(h, ei, wb2)


def _mm1_tc(x, W1):
  def body(x_ref, w_ref, o_ref):
    o_ref[...] = jnp.dot(x_ref[...], w_ref[...],
                         preferred_element_type=jnp.float32)

  return pl.pallas_call(
      body,
      grid=(10,),
      in_specs=[
          pl.BlockSpec((N // 10, D_IN), lambda i: (i, 0)),
          pl.BlockSpec((D_IN, H1), lambda i: (0, 0)),
      ],
      out_specs=pl.BlockSpec((N // 10, H1), lambda i: (i, 0)),
      out_shape=jax.ShapeDtypeStruct((N, H1), jnp.float32),
  )(x, W1)


def _relu_combine_tc(p):
  def body(p_ref, o_ref):
    o_ref[...] = jnp.maximum(p_ref[0] + p_ref[1], 0.0)

  blk = NPAD // 10
  return pl.pallas_call(
      body,
      grid=(10,),
      in_specs=[pl.BlockSpec((NC, blk, H1), lambda i: (0, i, 0))],
      out_specs=pl.BlockSpec((blk, H1), lambda i: (i, 0)),
      out_shape=jax.ShapeDtypeStruct((NPAD, H1), jnp.float32),
  )(p)


def _finalize_tc(q, eps, W2, W3):
  def body(q_ref, eps_ref, w2_ref, w3_ref, mu_ref, lv_ref, z_ref):
    s = q_ref[0] + q_ref[1]
    mu = jnp.dot(s, w2_ref[...], preferred_element_type=jnp.float32)
    lv = jnp.dot(s, w3_ref[...], preferred_element_type=jnp.float32)
    mu_ref[...] = mu
    lv_ref[...] = lv
    z_ref[...] = eps_ref[...] * jnp.exp(lv) + mu

  blk = NPAD // 10
  return pl.pallas_call(
      body,
      grid=(10,),
      in_specs=[
          pl.BlockSpec((NC, blk, H1), lambda i: (0, i, 0)),
          pl.BlockSpec((blk, H2), lambda i: (i, 0)),
          pl.BlockSpec((H1, H2), lambda i: (0, 0)),
          pl.BlockSpec((H1, H2), lambda i: (0, 0)),
      ],
      out_specs=[
          pl.BlockSpec((blk, H2), lambda i: (i, 0)),
          pl.BlockSpec((blk, H2), lambda i: (i, 0)),
          pl.BlockSpec((blk, H2), lambda i: (i, 0)),
      ],
      out_shape=[
          jax.ShapeDtypeStruct((N, H2), jnp.float32),
          jax.ShapeDtypeStruct((N, H2), jnp.float32),
          jax.ShapeDtypeStruct((N, H2), jnp.float32),
      ],
  )(q, eps, W2, W3)


def _gram_tc(z):
  BR = 256
  gr = pl.cdiv(N, BR)

  def body(zr_ref, zc_ref, o_ref):
    o_ref[...] = lax.dot_general(
        zr_ref[...], zc_ref[...],
        (((1,), (1,)), ((), ())),
        preferred_element_type=jnp.float32)

  return pl.pallas_call(
      body,
      grid=(gr,),
      in_specs=[
          pl.BlockSpec((BR, H2), lambda i: (i, 0)),
          pl.BlockSpec((N, H2), lambda i: (0, 0)),
      ],
      out_specs=pl.BlockSpec((BR, N), lambda i: (i, 0)),
      out_shape=jax.ShapeDtypeStruct((N, N), jnp.float32),
  )(z, z)


def kernel(x, edge_index, edge_weight, eps, W1, W2, W3):
  ei = edge_index.astype(jnp.int32)
  wb2 = jnp.broadcast_to(
      edge_weight.reshape(E // 8, 8)[:, :, None],
      (E // 8, 8, 16)).reshape(E // 8, 128)

  t1 = _mm1_tc(x, W1)
  p = _spmm_sc(t1, ei, wb2)
  h1 = _relu_combine_tc(p)
  q = _spmm_sc(h1, ei, wb2)
  mu, logvar, z = _finalize_tc(q, eps, W2, W3)
  adj = _gram_tc(z)
  return (adj, mu, logvar)


# speedup vs baseline: 1.3881x; 1.0106x over previous
import functools

import jax
import jax.numpy as jnp
from jax import lax
from jax.experimental import pallas as pl
from jax.experimental.pallas import tpu as pltpu
from jax.experimental.pallas import tpu_sc as plsc

N = 10000
E = 320000
D_IN = 128
H1 = 32
H2 = 16

NC = 2
NS = 16
NW = NC * NS
EPW = E // NW
CH = 400
NCH = EPW // CH
NPAD = 10240
RPT = NPAD // NS


WR = CH * 16 // 128


def _spmm_sc(h, ei, wb2):
  mesh = plsc.VectorSubcoreMesh(core_axis_name="c", subcore_axis_name="s")

  @functools.partial(
      pl.kernel,
      out_type=jax.ShapeDtypeStruct((NC, NPAD, H1), jnp.float32),
      mesh=mesh,
      compiler_params=pltpu.CompilerParams(
          use_tc_tiling_on_sc=False, needs_layout_passes=False),
      scratch_types=[
          [pltpu.VMEM((CH,), jnp.int32)] * 2,
          [pltpu.VMEM((CH,), jnp.int32)] * 2,
          [pltpu.VMEM((CH, H1), jnp.float32)] * 2,
          [pltpu.VMEM((WR, 128), jnp.float32)] * 2,
          pltpu.VMEM((CH, H1), jnp.float32),
          pltpu.VMEM((RPT, H1), jnp.float32),
          pltpu.VMEM((RPT, H1), jnp.float32),
          pltpu.VMEM_SHARED((NPAD, H1), jnp.float32),
          [pltpu.SemaphoreType.DMA] * 2,
          [pltpu.SemaphoreType.DMA] * 2,
      ],
  )
  def k(h_hbm, ei_hbm, wb_hbm, out_hbm,
        src_b, dst_b, rows_b, wb_b, wrows, stage_v, stage2_v, acc_sh,
        sems_e, sems_r):
    cid = lax.axis_index("c")
    sid = lax.axis_index("s")
    wid = cid * NS + sid
    base = wid * EPW

    zero16 = jnp.zeros((16,), jnp.float32)

    def zbody(i, carry):
      stage_v[i, pl.ds(0, 16)] = zero16
      stage_v[i, pl.ds(16, 16)] = zero16
      return carry

    lax.fori_loop(0, RPT, zbody, 0)
    pltpu.sync_copy(stage_v, acc_sh.at[pl.ds(sid * RPT, RPT)])


    def start_edges(c, par):
      off = base + c * CH
      wroff = off * 16 // 128
      pltpu.async_copy(ei_hbm.at[0, pl.ds(off, CH)], src_b[par], sems_e[par])
      pltpu.async_copy(ei_hbm.at[1, pl.ds(off, CH)], dst_b[par], sems_e[par])
      pltpu.async_copy(
          wb_hbm.at[pl.ds(wroff, WR)], wb_b[par], sems_e[par])

    def wait_edges(c, par):
      off = base + c * CH
      wroff = off * 16 // 128
      pltpu.make_async_copy(
          ei_hbm.at[0, pl.ds(off, CH)], src_b[par], sems_e[par]).wait()
      pltpu.make_async_copy(
          ei_hbm.at[1, pl.ds(off, CH)], dst_b[par], sems_e[par]).wait()
      pltpu.make_async_copy(
          wb_hbm.at[pl.ds(wroff, WR)], wb_b[par], sems_e[par]).wait()

    def start_rows(par):
      pltpu.async_copy(h_hbm.at[src_b[par]], rows_b[par], sems_r[par])

    def wait_rows(par):
      pltpu.make_async_copy(
          h_hbm.at[src_b[par]], rows_b[par], sems_r[par]).wait()

    def scale(par):
      def sbody(r, carry):
        for l8 in range(8):
          e = r * 8 + l8
          wv = wb_b[par][r, pl.ds(l8 * 16, 16)]
          wrows[e, pl.ds(0, 16)] = rows_b[par][e, pl.ds(0, 16)] * wv
          wrows[e, pl.ds(16, 16)] = rows_b[par][e, pl.ds(16, 16)] * wv
        return carry

      lax.fori_loop(0, WR, sbody, 0, unroll=2)

    start_edges(0, 0)
    start_edges(1, 1)
    wait_edges(0, 0)
    start_rows(0)

    def kbody(kk, carry):
      for par in range(2):
        c = 2 * kk + par

        @pl.when(c < NCH)
        def _():
          wait_rows(par)

          @pl.when(c + 1 < NCH)
          def _():
            wait_edges(c + 1, 1 - par)
            start_rows(1 - par)

          scale(par)
          pltpu.sync_copy(wrows, acc_sh.at[dst_b[par]], add=True)

          @pl.when(c + 2 < NCH)
          def _():
            start_edges(c + 2, par)

      return carry

    lax.fori_loop(0, (NCH + 1) // 2, kbody, 0)
    plsc.subcore_barrier()

    pltpu.sync_copy(acc_sh.at[pl.ds(sid * RPT, RPT)], stage_v)
    pltpu.sync_copy(stage_v, out_hbm.at[cid, pl.ds(sid * RPT, RPT)])

  return k(h, ei, wb2)


def _mm1_tc(x, W1):
  def body(x_ref, w_ref, o_ref):
    o_ref[...] = jnp.dot(x_ref[...], w_ref[...],
                         preferred_element_type=jnp.float32)

  return pl.pallas_call(
      body,
      out_shape=jax.ShapeDtypeStruct((N, H1), jnp.float32),
  )(x, W1)


def _relu_combine_tc(p):
  def body(p_ref, o_ref):
    o_ref[...] = jnp.maximum(p_ref[0] + p_ref[1], 0.0)

  return pl.pallas_call(
      body,
      out_shape=jax.ShapeDtypeStruct((NPAD, H1), jnp.float32),
  )(p)


def _finalize_tc(q, eps, W2, W3):
  def body(q_ref, eps_ref, w2_ref, w3_ref, mu_ref, lv_ref, z_ref):
    s = q_ref[0] + q_ref[1]
    mu = jnp.dot(s, w2_ref[...], preferred_element_type=jnp.float32)
    lv = jnp.dot(s, w3_ref[...], preferred_element_type=jnp.float32)
    mu_ref[...] = mu
    lv_ref[...] = lv
    z_ref[...] = eps_ref[...] * jnp.exp(lv) + mu

  blk = NPAD // 10
  return pl.pallas_call(
      body,
      grid=(10,),
      in_specs=[
          pl.BlockSpec((NC, blk, H1), lambda i: (0, i, 0)),
          pl.BlockSpec((blk, H2), lambda i: (i, 0)),
          pl.BlockSpec((H1, H2), lambda i: (0, 0)),
          pl.BlockSpec((H1, H2), lambda i: (0, 0)),
      ],
      out_specs=[
          pl.BlockSpec((blk, H2), lambda i: (i, 0)),
          pl.BlockSpec((blk, H2), lambda i: (i, 0)),
          pl.BlockSpec((blk, H2), lambda i: (i, 0)),
      ],
      out_shape=[
          jax.ShapeDtypeStruct((N, H2), jnp.float32),
          jax.ShapeDtypeStruct((N, H2), jnp.float32),
          jax.ShapeDtypeStruct((N, H2), jnp.float32),
      ],
  )(q, eps, W2, W3)


def _gram_tc(z):
  BR = 256
  gr = pl.cdiv(N, BR)

  def body(zr_ref, zc_ref, o_ref):
    o_ref[...] = lax.dot_general(
        zr_ref[...], zc_ref[...],
        (((1,), (1,)), ((), ())),
        preferred_element_type=jnp.float32)

  return pl.pallas_call(
      body,
      grid=(gr,),
      in_specs=[
          pl.BlockSpec((BR, H2), lambda i: (i, 0)),
          pl.BlockSpec((N, H2), lambda i: (0, 0)),
      ],
      out_specs=pl.BlockSpec((BR, N), lambda i: (i, 0)),
      out_shape=jax.ShapeDtypeStruct((N, N), jnp.float32),
  )(z, z)


def kernel(x, edge_index, edge_weight, eps, W1, W2, W3):
  ei = edge_index.astype(jnp.int32)
  wb2 = jnp.broadcast_to(
      edge_weight.reshape(E // 8, 8)[:, :, None],
      (E // 8, 8, 16)).reshape(E // 8, 128)

  t1 = _mm1_tc(x, W1)
  p = _spmm_sc(t1, ei, wb2)
  h1 = _relu_combine_tc(p)
  q = _spmm_sc(h1, ei, wb2)
  mu, logvar, z = _finalize_tc(q, eps, W2, W3)
  adj = _gram_tc(z)
  return (adj, mu, logvar)


# speedup vs baseline: 1.6072x; 1.1578x over previous
import functools

import jax
import jax.numpy as jnp
from jax import lax
from jax.experimental import pallas as pl
from jax.experimental.pallas import tpu as pltpu
from jax.experimental.pallas import tpu_sc as plsc

N = 10000
E = 320000
D_IN = 128
H1 = 32
H2 = 16

NC = 2
NS = 16
NW = NC * NS
EPW = E // NW
CH = 400
NCH = EPW // CH
NPAD = 10240
RPT = NPAD // NS


WR = CH * 16 // 128


def _spmm_sc(h, ei, wb2):
  mesh = plsc.VectorSubcoreMesh(core_axis_name="c", subcore_axis_name="s")

  @functools.partial(
      pl.kernel,
      out_type=jax.ShapeDtypeStruct((NC, NPAD, H1), jnp.float32),
      mesh=mesh,
      compiler_params=pltpu.CompilerParams(
          use_tc_tiling_on_sc=False, needs_layout_passes=False),
      scratch_types=[
          [pltpu.VMEM((CH,), jnp.int32)] * 2,
          [pltpu.VMEM((CH,), jnp.int32)] * 2,
          [pltpu.VMEM((CH, H1), jnp.float32)] * 2,
          [pltpu.VMEM((WR, 128), jnp.float32)] * 2,
          pltpu.VMEM((RPT, H1), jnp.float32),
          pltpu.VMEM((RPT, H1), jnp.float32),
          pltpu.VMEM_SHARED((NPAD, H1), jnp.float32),
          [pltpu.SemaphoreType.DMA] * 2,
          [pltpu.SemaphoreType.DMA] * 2,
      ],
  )
  def k(h_hbm, ei_hbm, wb_hbm, out_hbm,
        src_b, dst_b, rows_b, wb_b, stage_v, stage2_v, acc_sh,
        sems_e, sems_r):
    cid = lax.axis_index("c")
    sid = lax.axis_index("s")
    wid = cid * NS + sid
    base = wid * EPW

    zero16 = jnp.zeros((16,), jnp.float32)

    def zbody(i, carry):
      stage_v[i, pl.ds(0, 16)] = zero16
      stage_v[i, pl.ds(16, 16)] = zero16
      return carry

    lax.fori_loop(0, RPT, zbody, 0)
    pltpu.sync_copy(stage_v, acc_sh.at[pl.ds(sid * RPT, RPT)])


    def start_edges(c, par):
      off = base + c * CH
      wroff = off * 16 // 128
      pltpu.async_copy(ei_hbm.at[0, pl.ds(off, CH)], src_b[par], sems_e[par])
      pltpu.async_copy(ei_hbm.at[1, pl.ds(off, CH)], dst_b[par], sems_e[par])
      pltpu.async_copy(
          wb_hbm.at[pl.ds(wroff, WR)], wb_b[par], sems_e[par])

    def wait_edges(c, par):
      off = base + c * CH
      wroff = off * 16 // 128
      pltpu.make_async_copy(
          ei_hbm.at[0, pl.ds(off, CH)], src_b[par], sems_e[par]).wait()
      pltpu.make_async_copy(
          ei_hbm.at[1, pl.ds(off, CH)], dst_b[par], sems_e[par]).wait()
      pltpu.make_async_copy(
          wb_hbm.at[pl.ds(wroff, WR)], wb_b[par], sems_e[par]).wait()

    def start_rows(par):
      pltpu.async_copy(h_hbm.at[src_b[par]], rows_b[par], sems_r[par])

    def wait_rows(par):
      pltpu.make_async_copy(
          h_hbm.at[src_b[par]], rows_b[par], sems_r[par]).wait()

    def scale(par):
      for e in range(CH):
        wv = wb_b[par][e // 8, pl.ds((e % 8) * 16, 16)]
        rows_b[par][e, pl.ds(0, 16)] = rows_b[par][e, pl.ds(0, 16)] * wv
        rows_b[par][e, pl.ds(16, 16)] = rows_b[par][e, pl.ds(16, 16)] * wv

    start_edges(0, 0)
    start_edges(1, 1)
    wait_edges(0, 0)
    start_rows(0)

    def kbody(kk, carry):
      for par in range(2):
        c = 2 * kk + par

        @pl.when(c < NCH)
        def _():
          wait_rows(par)

          @pl.when(c + 1 < NCH)
          def _():
            wait_edges(c + 1, 1 - par)
            start_rows(1 - par)

          scale(par)
          pltpu.sync_copy(rows_b[par], acc_sh.at[dst_b[par]], add=True)

          @pl.when(c + 2 < NCH)
          def _():
            start_edges(c + 2, par)

      return carry

    lax.fori_loop(0, (NCH + 1) // 2, kbody, 0)
    plsc.subcore_barrier()

    pltpu.sync_copy(acc_sh.at[pl.ds(sid * RPT, RPT)], stage_v)
    pltpu.sync_copy(stage_v, out_hbm.at[cid, pl.ds(sid * RPT, RPT)])

  return k(h, ei, wb2)


def _mm1_tc(x, W1):
  def body(x_ref, w_ref, o_ref):
    o_ref[...] = jnp.dot(x_ref[...], w_ref[...],
                         preferred_element_type=jnp.float32)

  return pl.pallas_call(
      body,
      out_shape=jax.ShapeDtypeStruct((N, H1), jnp.float32),
  )(x, W1)


def _relu_combine_tc(p):
  def body(p_ref, o_ref):
    o_ref[...] = jnp.maximum(p_ref[0] + p_ref[1], 0.0)

  return pl.pallas_call(
      body,
      out_shape=jax.ShapeDtypeStruct((NPAD, H1), jnp.float32),
  )(p)


def _finalize_tc(q, eps, W2, W3):
  def body(q_ref, eps_ref, w2_ref, w3_ref, mu_ref, lv_ref, z_ref):
    s = q_ref[0] + q_ref[1]
    mu = jnp.dot(s, w2_ref[...], preferred_element_type=jnp.float32)
    lv = jnp.dot(s, w3_ref[...], preferred_element_type=jnp.float32)
    mu_ref[...] = mu
    lv_ref[...] = lv
    z_ref[...] = eps_ref[...] * jnp.exp(lv) + mu

  blk = NPAD // 10
  return pl.pallas_call(
      body,
      grid=(10,),
      in_specs=[
          pl.BlockSpec((NC, blk, H1), lambda i: (0, i, 0)),
          pl.BlockSpec((blk, H2), lambda i: (i, 0)),
          pl.BlockSpec((H1, H2), lambda i: (0, 0)),
          pl.BlockSpec((H1, H2), lambda i: (0, 0)),
      ],
      out_specs=[
          pl.BlockSpec((blk, H2), lambda i: (i, 0)),
          pl.BlockSpec((blk, H2), lambda i: (i, 0)),
          pl.BlockSpec((blk, H2), lambda i: (i, 0)),
      ],
      out_shape=[
          jax.ShapeDtypeStruct((N, H2), jnp.float32),
          jax.ShapeDtypeStruct((N, H2), jnp.float32),
          jax.ShapeDtypeStruct((N, H2), jnp.float32),
      ],
  )(q, eps, W2, W3)


def _gram_tc(z):
  BR = 256
  gr = pl.cdiv(N, BR)

  def body(zr_ref, zc_ref, o_ref):
    o_ref[...] = lax.dot_general(
        zr_ref[...], zc_ref[...],
        (((1,), (1,)), ((), ())),
        preferred_element_type=jnp.float32)

  return pl.pallas_call(
      body,
      grid=(gr,),
      in_specs=[
          pl.BlockSpec((BR, H2), lambda i: (i, 0)),
          pl.BlockSpec((N, H2), lambda i: (0, 0)),
      ],
      out_specs=pl.BlockSpec((BR, N), lambda i: (i, 0)),
      out_shape=jax.ShapeDtypeStruct((N, N), jnp.float32),
  )(z, z)


def kernel(x, edge_index, edge_weight, eps, W1, W2, W3):
  ei = edge_index.astype(jnp.int32)
  wb2 = jnp.broadcast_to(
      edge_weight.reshape(E // 8, 8)[:, :, None],
      (E // 8, 8, 16)).reshape(E // 8, 128)

  t1 = _mm1_tc(x, W1)
  p = _spmm_sc(t1, ei, wb2)
  h1 = _relu_combine_tc(p)
  q = _spmm_sc(h1, ei, wb2)
  mu, logvar, z = _finalize_tc(q, eps, W2, W3)
  adj = _gram_tc(z)
  return (adj, mu, logvar)


# speedup vs baseline: 1.6090x; 1.0012x over previous
import functools

import jax
import jax.numpy as jnp
from jax import lax
from jax.experimental import pallas as pl
from jax.experimental.pallas import tpu as pltpu
from jax.experimental.pallas import tpu_sc as plsc

N = 10000
E = 320000
D_IN = 128
H1 = 32
H2 = 16

NC = 2
NS = 16
NW = NC * NS
EPW = E // NW
CH = 400
NCH = EPW // CH
NPAD = 10240
RPT = NPAD // NS


WR = CH * 16 // 128


def _spmm_sc(h, ei, wb2):
  mesh = plsc.VectorSubcoreMesh(core_axis_name="c", subcore_axis_name="s")

  @functools.partial(
      pl.kernel,
      out_type=jax.ShapeDtypeStruct((NC, NPAD, H1), jnp.float32),
      mesh=mesh,
      compiler_params=pltpu.CompilerParams(
          use_tc_tiling_on_sc=False, needs_layout_passes=False),
      scratch_types=[
          [pltpu.VMEM((CH,), jnp.int32)] * 2,
          [pltpu.VMEM((CH,), jnp.int32)] * 2,
          [pltpu.VMEM((CH, H1), jnp.float32)] * 2,
          [pltpu.VMEM((WR, 128), jnp.float32)] * 2,
          pltpu.VMEM((CH, H1), jnp.float32),
          pltpu.VMEM((RPT, H1), jnp.float32),
          pltpu.VMEM((RPT, H1), jnp.float32),
          pltpu.VMEM_SHARED((NPAD, H1), jnp.float32),
          [pltpu.SemaphoreType.DMA] * 2,
          [pltpu.SemaphoreType.DMA] * 2,
      ],
  )
  def k(h_hbm, ei_hbm, wb_hbm, out_hbm,
        src_b, dst_b, rows_b, wb_b, wrows, stage_v, stage2_v, acc_sh,
        sems_e, sems_r):
    cid = lax.axis_index("c")
    sid = lax.axis_index("s")
    wid = cid * NS + sid
    base = wid * EPW

    zero16 = jnp.zeros((16,), jnp.float32)

    def zbody(i, carry):
      stage_v[i, pl.ds(0, 16)] = zero16
      stage_v[i, pl.ds(16, 16)] = zero16
      return carry

    lax.fori_loop(0, RPT, zbody, 0)
    pltpu.sync_copy(stage_v, acc_sh.at[pl.ds(sid * RPT, RPT)])


    def start_edges(c, par):
      off = base + c * CH
      wroff = off * 16 // 128
      pltpu.async_copy(ei_hbm.at[0, pl.ds(off, CH)], src_b[par], sems_e[par])
      pltpu.async_copy(ei_hbm.at[1, pl.ds(off, CH)], dst_b[par], sems_e[par])
      pltpu.async_copy(
          wb_hbm.at[pl.ds(wroff, WR)], wb_b[par], sems_e[par])

    def wait_edges(c, par):
      off = base + c * CH
      wroff = off * 16 // 128
      pltpu.make_async_copy(
          ei_hbm.at[0, pl.ds(off, CH)], src_b[par], sems_e[par]).wait()
      pltpu.make_async_copy(
          ei_hbm.at[1, pl.ds(off, CH)], dst_b[par], sems_e[par]).wait()
      pltpu.make_async_copy(
          wb_hbm.at[pl.ds(wroff, WR)], wb_b[par], sems_e[par]).wait()

    def start_rows(par):
      pltpu.async_copy(h_hbm.at[src_b[par]], rows_b[par], sems_r[par])

    def wait_rows(par):
      pltpu.make_async_copy(
          h_hbm.at[src_b[par]], rows_b[par], sems_r[par]).wait()

    def scale(par):
      for e in range(CH):
        wv = wb_b[par][e // 8, pl.ds((e % 8) * 16, 16)]
        wrows[e, pl.ds(0, 16)] = rows_b[par][e, pl.ds(0, 16)] * wv
        wrows[e, pl.ds(16, 16)] = rows_b[par][e, pl.ds(16, 16)] * wv

    start_edges(0, 0)
    start_edges(1, 1)
    wait_edges(0, 0)
    start_rows(0)

    def kbody(kk, carry):
      for par in range(2):
        c = 2 * kk + par

        @pl.when(c < NCH)
        def _():
          wait_rows(par)

          @pl.when(c + 1 < NCH)
          def _():
            wait_edges(c + 1, 1 - par)
            start_rows(1 - par)

          scale(par)
          pltpu.sync_copy(wrows, acc_sh.at[dst_b[par]], add=True)

          @pl.when(c + 2 < NCH)
          def _():
            start_edges(c + 2, par)

      return carry

    lax.fori_loop(0, (NCH + 1) // 2, kbody, 0)
    plsc.subcore_barrier()

    pltpu.sync_copy(acc_sh.at[pl.ds(sid * RPT, RPT)], stage_v)
    pltpu.sync_copy(stage_v, out_hbm.at[cid, pl.ds(sid * RPT, RPT)])

  return k(h, ei, wb2)


def _mm1_tc(x, W1):
  def body(x_ref, w_ref, o_ref):
    o_ref[...] = jnp.dot(x_ref[...], w_ref[...],
                         preferred_element_type=jnp.float32)

  return pl.pallas_call(
      body,
      out_shape=jax.ShapeDtypeStruct((N, H1), jnp.float32),
  )(x, W1)


def _relu_combine_tc(p):
  def body(p_ref, o_ref):
    o_ref[...] = jnp.maximum(p_ref[0] + p_ref[1], 0.0)

  return pl.pallas_call(
      body,
      out_shape=jax.ShapeDtypeStruct((NPAD, H1), jnp.float32),
  )(p)


def _finalize_tc(q, eps, W2, W3):
  def body(q_ref, eps_ref, w2_ref, w3_ref, mu_ref, lv_ref, z_ref):
    s = q_ref[0] + q_ref[1]
    mu = jnp.dot(s, w2_ref[...], preferred_element_type=jnp.float32)
    lv = jnp.dot(s, w3_ref[...], preferred_element_type=jnp.float32)
    mu_ref[...] = mu
    lv_ref[...] = lv
    z_ref[...] = eps_ref[...] * jnp.exp(lv) + mu

  blk = NPAD // 10
  return pl.pallas_call(
      body,
      grid=(10,),
      in_specs=[
          pl.BlockSpec((NC, blk, H1), lambda i: (0, i, 0)),
          pl.BlockSpec((blk, H2), lambda i: (i, 0)),
          pl.BlockSpec((H1, H2), lambda i: (0, 0)),
          pl.BlockSpec((H1, H2), lambda i: (0, 0)),
      ],
      out_specs=[
          pl.BlockSpec((blk, H2), lambda i: (i, 0)),
          pl.BlockSpec((blk, H2), lambda i: (i, 0)),
          pl.BlockSpec((blk, H2), lambda i: (i, 0)),
      ],
      out_shape=[
          jax.ShapeDtypeStruct((N, H2), jnp.float32),
          jax.ShapeDtypeStruct((N, H2), jnp.float32),
          jax.ShapeDtypeStruct((N, H2), jnp.float32),
      ],
  )(q, eps, W2, W3)


def _gram_tc(z):
  BR = 256
  gr = pl.cdiv(N, BR)

  def body(zr_ref, zc_ref, o_ref):
    o_ref[...] = lax.dot_general(
        zr_ref[...], zc_ref[...],
        (((1,), (1,)), ((), ())),
        preferred_element_type=jnp.float32)

  return pl.pallas_call(
      body,
      grid=(gr,),
      in_specs=[
          pl.BlockSpec((BR, H2), lambda i: (i, 0)),
          pl.BlockSpec((N, H2), lambda i: (0, 0)),
      ],
      out_specs=pl.BlockSpec((BR, N), lambda i: (i, 0)),
      out_shape=jax.ShapeDtypeStruct((N, N), jnp.float32),
  )(z, z)


def kernel(x, edge_index, edge_weight, eps, W1, W2, W3):
  ei = edge_index.astype(jnp.int32)
  wb2 = jnp.broadcast_to(
      edge_weight.reshape(E // 8, 8)[:, :, None],
      (E // 8, 8, 16)).reshape(E // 8, 128)

  t1 = _mm1_tc(x, W1)
  p = _spmm_sc(t1, ei, wb2)
  h1 = _relu_combine_tc(p)
  q = _spmm_sc(h1, ei, wb2)
  mu, logvar, z = _finalize_tc(q, eps, W2, W3)
  adj = _gram_tc(z)
  return (adj, mu, logvar)


# speedup vs baseline: 1.6170x; 1.0049x over previous
import functools

import jax
import jax.numpy as jnp
from jax import lax
from jax.experimental import pallas as pl
from jax.experimental.pallas import tpu as pltpu
from jax.experimental.pallas import tpu_sc as plsc

N = 10000
E = 320000
D_IN = 128
H1 = 32
H2 = 16

NC = 2
NS = 16
NW = NC * NS
EPW = E // NW
CH = 400
NCH = EPW // CH
NPAD = 10240
RPT = NPAD // NS


WR = CH * 16 // 128


def _spmm_sc(h, ei, wb2):
  mesh = plsc.VectorSubcoreMesh(core_axis_name="c", subcore_axis_name="s")

  @functools.partial(
      pl.kernel,
      out_type=jax.ShapeDtypeStruct((NC, NPAD, H1), jnp.float32),
      mesh=mesh,
      compiler_params=pltpu.CompilerParams(
          use_tc_tiling_on_sc=False, needs_layout_passes=False),
      scratch_types=[
          [pltpu.VMEM((CH,), jnp.int32)] * 2,
          [pltpu.VMEM((CH,), jnp.int32)] * 2,
          [pltpu.VMEM((CH, H1), jnp.float32)] * 2,
          [pltpu.VMEM((WR, 128), jnp.float32)] * 2,
          pltpu.VMEM((CH, H1), jnp.float32),
          pltpu.VMEM((RPT, H1), jnp.float32),
          pltpu.VMEM((RPT, H1), jnp.float32),
          pltpu.VMEM_SHARED((NPAD, H1), jnp.float32),
          [pltpu.SemaphoreType.DMA] * 2,
          [pltpu.SemaphoreType.DMA] * 2,
      ],
  )
  def k(h_hbm, ei_hbm, wb_hbm, out_hbm,
        src_b, dst_b, rows_b, wb_b, wrows, stage_v, stage2_v, acc_sh,
        sems_e, sems_r):
    cid = lax.axis_index("c")
    sid = lax.axis_index("s")
    wid = cid * NS + sid
    base = wid * EPW

    def start_edges(c, par):
      off = base + c * CH
      wroff = off * 16 // 128
      pltpu.async_copy(ei_hbm.at[0, pl.ds(off, CH)], src_b[par], sems_e[par])
      pltpu.async_copy(ei_hbm.at[1, pl.ds(off, CH)], dst_b[par], sems_e[par])
      pltpu.async_copy(
          wb_hbm.at[pl.ds(wroff, WR)], wb_b[par], sems_e[par])

    def wait_edges(c, par):
      off = base + c * CH
      wroff = off * 16 // 128
      pltpu.make_async_copy(
          ei_hbm.at[0, pl.ds(off, CH)], src_b[par], sems_e[par]).wait()
      pltpu.make_async_copy(
          ei_hbm.at[1, pl.ds(off, CH)], dst_b[par], sems_e[par]).wait()
      pltpu.make_async_copy(
          wb_hbm.at[pl.ds(wroff, WR)], wb_b[par], sems_e[par]).wait()

    def start_rows(par):
      pltpu.async_copy(h_hbm.at[src_b[par]], rows_b[par], sems_r[par])

    def wait_rows(par):
      pltpu.make_async_copy(
          h_hbm.at[src_b[par]], rows_b[par], sems_r[par]).wait()

    def scale(par):
      for e in range(CH):
        wv = wb_b[par][e // 8, pl.ds((e % 8) * 16, 16)]
        wrows[e, pl.ds(0, 16)] = rows_b[par][e, pl.ds(0, 16)] * wv
        wrows[e, pl.ds(16, 16)] = rows_b[par][e, pl.ds(16, 16)] * wv

    start_edges(0, 0)
    start_edges(1, 1)

    zero16 = jnp.zeros((16,), jnp.float32)

    def zbody(i, carry):
      stage_v[i, pl.ds(0, 16)] = zero16
      stage_v[i, pl.ds(16, 16)] = zero16
      return carry

    lax.fori_loop(0, RPT, zbody, 0)
    pltpu.sync_copy(stage_v, acc_sh.at[pl.ds(sid * RPT, RPT)])
    plsc.subcore_barrier()

    wait_edges(0, 0)
    start_rows(0)

    def kbody(kk, carry):
      for par in range(2):
        c = 2 * kk + par

        @pl.when(c < NCH)
        def _():
          wait_rows(par)

          @pl.when(c + 1 < NCH)
          def _():
            wait_edges(c + 1, 1 - par)
            start_rows(1 - par)

          scale(par)
          pltpu.sync_copy(wrows, acc_sh.at[dst_b[par]], add=True)

          @pl.when(c + 2 < NCH)
          def _():
            start_edges(c + 2, par)

      return carry

    lax.fori_loop(0, (NCH + 1) // 2, kbody, 0)
    plsc.subcore_barrier()

    pltpu.sync_copy(acc_sh.at[pl.ds(sid * RPT, RPT)],
                    out_hbm.at[cid, pl.ds(sid * RPT, RPT)])

  return k(h, ei, wb2)


def _mm1_tc(x, W1):
  def body(x_ref, w_ref, o_ref):
    o_ref[...] = jnp.dot(x_ref[...], w_ref[...],
                         preferred_element_type=jnp.float32)

  return pl.pallas_call(
      body,
      out_shape=jax.ShapeDtypeStruct((N, H1), jnp.float32),
  )(x, W1)


def _relu_combine_tc(p):
  def body(p_ref, o_ref):
    o_ref[...] = jnp.maximum(p_ref[0] + p_ref[1], 0.0)

  return pl.pallas_call(
      body,
      out_shape=jax.ShapeDtypeStruct((NPAD, H1), jnp.float32),
  )(p)


def _finalize_tc(q, eps, W2, W3):
  def body(q_ref, eps_ref, w2_ref, w3_ref, mu_ref, lv_ref, z_ref):
    s = q_ref[0] + q_ref[1]
    mu = jnp.dot(s, w2_ref[...], preferred_element_type=jnp.float32)
    lv = jnp.dot(s, w3_ref[...], preferred_element_type=jnp.float32)
    mu_ref[...] = mu
    lv_ref[...] = lv
    z_ref[...] = eps_ref[...] * jnp.exp(lv) + mu

  blk = NPAD // 10
  return pl.pallas_call(
      body,
      grid=(10,),
      in_specs=[
          pl.BlockSpec((NC, blk, H1), lambda i: (0, i, 0)),
          pl.BlockSpec((blk, H2), lambda i: (i, 0)),
          pl.BlockSpec((H1, H2), lambda i: (0, 0)),
          pl.BlockSpec((H1, H2), lambda i: (0, 0)),
      ],
      out_specs=[
          pl.BlockSpec((blk, H2), lambda i: (i, 0)),
          pl.BlockSpec((blk, H2), lambda i: (i, 0)),
          pl.BlockSpec((blk, H2), lambda i: (i, 0)),
      ],
      out_shape=[
          jax.ShapeDtypeStruct((N, H2), jnp.float32),
          jax.ShapeDtypeStruct((N, H2), jnp.float32),
          jax.ShapeDtypeStruct((N, H2), jnp.float32),
      ],
  )(q, eps, W2, W3)


def _gram_tc(z):
  BR = 256
  gr = pl.cdiv(N, BR)

  def body(zr_ref, zc_ref, o_ref):
    o_ref[...] = lax.dot_general(
        zr_ref[...], zc_ref[...],
        (((1,), (1,)), ((), ())),
        preferred_element_type=jnp.float32)

  return pl.pallas_call(
      body,
      grid=(gr,),
      in_specs=[
          pl.BlockSpec((BR, H2), lambda i: (i, 0)),
          pl.BlockSpec((N, H2), lambda i: (0, 0)),
      ],
      out_specs=pl.BlockSpec((BR, N), lambda i: (i, 0)),
      out_shape=jax.ShapeDtypeStruct((N, N), jnp.float32),
  )(z, z)


def kernel(x, edge_index, edge_weight, eps, W1, W2, W3):
  ei = edge_index.astype(jnp.int32)
  wb2 = jnp.broadcast_to(
      edge_weight.reshape(E // 8, 8)[:, :, None],
      (E // 8, 8, 16)).reshape(E // 8, 128)

  t1 = _mm1_tc(x, W1)
  p = _spmm_sc(t1, ei, wb2)
  h1 = _relu_combine_tc(p)
  q = _spmm_sc(h1, ei, wb2)
  mu, logvar, z = _finalize_tc(q, eps, W2, W3)
  adj = _gram_tc(z)
  return (adj, mu, logvar)
